# baseline (device time: 92773 ns/iter reference)
import jax
import jax.numpy as jnp
from jax import lax
from jax.experimental import pallas as pl
from jax.experimental.pallas import tpu as pltpu

N_DEV = 4
SQ = 256
SQ_G = SQ * N_DEV
D = 1024
HQ = 8
DH = 128
SKV = 4096
KV_T = 1024
N_KV_T = SKV // KV_T
SCALE = 0.08838834764831843

_MESH = pl.DeviceIdType.MESH


def kernel(x, Wq, Wo, K_ext, V_ext):
    x2 = x.reshape(SQ, D)
    K3 = K_ext.reshape(SKV, HQ, DH)
    V3 = V_ext.reshape(SKV, HQ, DH)

    def body(x_ref, wq_ref, wo_ref, k_ref, v_ref, out_ref,
             q_comm, k_buf, v_buf, o_part, l_scr,
             o_comm, l_comm,
             kv_sems, q_send, q_recv,
             o_send, o_recv, l_send, l_recv):
        my = lax.axis_index("i")

        bar = pltpu.get_barrier_semaphore()
        for d in range(N_DEV):
            @pl.when(my != d)
            def _(d=d):
                pl.semaphore_signal(bar, inc=1, device_id=(d,),
                                    device_id_type=_MESH)
        pl.semaphore_wait(bar, N_DEV - 1)

        q_local = lax.dot_general(x_ref[...], wq_ref[...],
                                  (((1,), (0,)), ((), ())),
                                  preferred_element_type=jnp.float32)
        q_comm[pl.ds(my * SQ, SQ), :] = q_local * SCALE

        q_rdmas = []
        for d in range(N_DEV):
            r = pltpu.make_async_remote_copy(
                src_ref=q_comm.at[pl.ds(my * SQ, SQ)],
                dst_ref=q_comm.at[pl.ds(my * SQ, SQ)],
                send_sem=q_send.at[d],
                recv_sem=q_recv.at[my],
                device_id=(d,),
                device_id_type=_MESH,
            )
            q_rdmas.append(r)

            @pl.when(my != d)
            def _(r=r):
                r.start()

        def kv_fetch(j, slot):
            ds = []
            for h in range(HQ):
                ds.append(pltpu.make_async_copy(
                    k_ref.at[pl.ds(j * KV_T, KV_T), h], k_buf.at[slot, h],
                    kv_sems.at[0, slot]))
                ds.append(pltpu.make_async_copy(
                    v_ref.at[pl.ds(j * KV_T, KV_T), h],
                    v_buf.at[slot, h, slice(None), slice(0, DH)],
                    kv_sems.at[1, slot]))
            return ds

        pending = kv_fetch(0, 0)
        for dma in pending:
            dma.start()

        for sl in range(2):
            for h in range(HQ):
                v_buf[sl, h, :, DH:DH + 1] = jnp.ones((KV_T, 1), jnp.float32)

        def o_head_rdma(d, h):
            cols = slice(h * DH, (h + 1) * DH)
            return pltpu.make_async_remote_copy(
                src_ref=o_part.at[pl.ds(d * SQ, SQ), cols],
                dst_ref=o_comm.at[my, :, cols],
                send_sem=o_send.at[d, h], recv_sem=o_recv.at[my, h],
                device_id=(d,), device_id_type=_MESH)

        o_rdmas = []
        for j in range(N_KV_T):
            slot = j % 2
            for dma in pending:
                dma.wait()
            if j + 1 < N_KV_T:
                pending = kv_fetch(j + 1, (j + 1) % 2)
                for dma in pending:
                    dma.start()
            last = j == N_KV_T - 1
            if j == 0:
                for t in range(N_DEV):
                    b = (my + t) % N_DEV
                    rows = pl.ds(b * SQ, SQ)
                    if t > 0:
                        r = pltpu.make_async_remote_copy(
                            src_ref=q_comm.at[rows],
                            dst_ref=q_comm.at[rows],
                            send_sem=q_send.at[b],
                            recv_sem=q_recv.at[b],
                            device_id=(0,),
                            device_id_type=_MESH,
                        )
                        r.wait_recv()
                    for h in range(HQ):
                        cols = slice(h * DH, (h + 1) * DH)
                        qh = q_comm[rows, cols]
                        kh = k_buf[slot, h]
                        s_h = lax.dot_general(
                            qh, kh, (((1,), (1,)), ((), ())),
                            preferred_element_type=jnp.float32)
                        p = jnp.exp(s_h)
                        pv_l = lax.dot_general(
                            p, v_buf[slot, h], (((1,), (0,)), ((), ())),
                            preferred_element_type=jnp.float32)
                        l_scr[rows, h:h + 1] = pv_l[:, DH:DH + 1]
                        o_part[rows, cols] = pv_l[:, :DH]
                continue
            for h in range(HQ):
                cols = slice(h * DH, (h + 1) * DH)
                qh = q_comm[:, cols]
                kh = k_buf[slot, h]
                s_h = lax.dot_general(
                    qh, kh, (((1,), (1,)), ((), ())),
                    preferred_element_type=jnp.float32)
                p = jnp.exp(s_h)
                pv_l = lax.dot_general(
                    p, v_buf[slot, h], (((1,), (0,)), ((), ())),
                    preferred_element_type=jnp.float32)
                l_scr[:, h:h + 1] = l_scr[:, h:h + 1] + pv_l[:, DH:DH + 1]
                o_part[:, cols] = o_part[:, cols] + pv_l[:, :DH]
                if last:
                    for d in range(N_DEV):
                        r = o_head_rdma(d, h)
                        o_rdmas.append(r)

                        @pl.when(my != d)
                        def _(r=r):
                            r.start()

        l_rdmas = []
        for d in range(N_DEV):
            rl = pltpu.make_async_remote_copy(
                src_ref=l_scr.at[pl.ds(d * SQ, SQ)],
                dst_ref=l_comm.at[my],
                send_sem=l_send.at[d], recv_sem=l_recv.at[my],
                device_id=(d,), device_id_type=_MESH)
            l_rdmas.append(rl)

            @pl.when(my != d)
            def _(rl=rl):
                rl.start()

        for s in range(N_DEV):
            rl = pltpu.make_async_remote_copy(
                src_ref=l_scr.at[pl.ds(s * SQ, SQ)],
                dst_ref=l_comm.at[s],
                send_sem=l_send.at[s], recv_sem=l_recv.at[s],
                device_id=(s,), device_id_type=_MESH)
            waiters = [rl]
            for h in range(HQ):
                cols = slice(h * DH, (h + 1) * DH)
                waiters.append(pltpu.make_async_remote_copy(
                    src_ref=o_part.at[pl.ds(s * SQ, SQ), cols],
                    dst_ref=o_comm.at[s, :, cols],
                    send_sem=o_send.at[s, h], recv_sem=o_recv.at[s, h],
                    device_id=(s,), device_id_type=_MESH))

            @pl.when(my != s)
            def _(ws=tuple(waiters)):
                for w in ws:
                    w.wait_recv()

        own_o = o_part[pl.ds(my * SQ, SQ), :]
        own_l = l_scr[pl.ds(my * SQ, SQ), :]
        o_tot = own_o
        l_tot = own_l
        for s in range(N_DEV):
            is_own = my == s
            zero_o = jnp.zeros((SQ, D), jnp.float32)
            zero_l = jnp.zeros((SQ, HQ), jnp.float32)
            o_tot = o_tot + jnp.where(is_own, zero_o, o_comm[s])
            l_tot = l_tot + jnp.where(is_own, zero_l, l_comm[s])

        head_outs = []
        for h in range(HQ):
            cols = slice(h * DH, (h + 1) * DH)
            head_outs.append(o_tot[:, cols] / l_tot[:, h:h + 1])
        attn = jnp.concatenate(head_outs, axis=1)

        out_ref[...] = lax.dot_general(attn, wo_ref[...],
                                       (((1,), (0,)), ((), ())),
                                       preferred_element_type=jnp.float32)

        for d in range(N_DEV):
            @pl.when(my != d)
            def _(q=q_rdmas[d], rl=l_rdmas[d]):
                q.wait_send()
                rl.wait_send()
        for i, r in enumerate(o_rdmas):
            @pl.when(my != (i % N_DEV))
            def _(r=r):
                r.wait_send()

    out = pl.pallas_call(
        body,
        out_shape=jax.ShapeDtypeStruct((SQ, D), jnp.float32),
        in_specs=[
            pl.BlockSpec(memory_space=pltpu.VMEM),
            pl.BlockSpec(memory_space=pltpu.VMEM),
            pl.BlockSpec(memory_space=pltpu.VMEM),
            pl.BlockSpec(memory_space=pl.ANY),
            pl.BlockSpec(memory_space=pl.ANY),
        ],
        out_specs=pl.BlockSpec(memory_space=pltpu.VMEM),
        scratch_shapes=[
            pltpu.VMEM((SQ_G, D), jnp.float32),
            pltpu.VMEM((2, HQ, KV_T, DH), jnp.float32),
            pltpu.VMEM((2, HQ, KV_T, DH + 128), jnp.float32),
            pltpu.VMEM((SQ_G, D), jnp.float32),
            pltpu.VMEM((SQ_G, HQ), jnp.float32),
            pltpu.VMEM((N_DEV, SQ, D), jnp.float32),
            pltpu.VMEM((N_DEV, SQ, HQ), jnp.float32),
            pltpu.SemaphoreType.DMA((2, 2)),
            pltpu.SemaphoreType.DMA((N_DEV,)),
            pltpu.SemaphoreType.DMA((N_DEV,)),
            pltpu.SemaphoreType.DMA((N_DEV, HQ)),
            pltpu.SemaphoreType.DMA((N_DEV, HQ)),
            pltpu.SemaphoreType.DMA((N_DEV,)),
            pltpu.SemaphoreType.DMA((N_DEV,)),
        ],
        compiler_params=pltpu.CompilerParams(
            collective_id=0, vmem_limit_bytes=50 * 1024 * 1024),
    )(x2, Wq, Wo, K3, V3)

    return out.reshape(1, SQ, D)
